# baseline (speedup 1.0000x reference)
import jax
import jax.numpy as jnp
from jax.experimental import pallas as pl
from jax.experimental.pallas import tpu as pltpu

N_EMB = 1024
EMB_DIM = 64
B = 16
HW = 1024
N_TOK = B * HW
COMMITMENT_COST = 0.25


def _vq_block(z_ref, emb_ref, dist_ref, idx_ref, zq_ref, loss_ref, perp_ref,
              counts_acc, loss_acc):
    b = pl.program_id(0)

    x_slab = z_ref[0]
    emb = emb_ref[...]

    x2 = jnp.sum(x_slab * x_slab, axis=0, keepdims=True).T
    e2 = jnp.sum(emb * emb, axis=1, keepdims=True).T
    xe = jax.lax.dot_general(
        x_slab, emb, (((0,), (1,)), ((), ())),
        preferred_element_type=jnp.float32)
    dist = (x2 + e2) - 2.0 * xe
    dist_ref[...] = dist

    min_d = jnp.min(dist, axis=1, keepdims=True)
    iota = jax.lax.broadcasted_iota(jnp.int32, (HW, N_EMB), 1)
    idx = jnp.min(jnp.where(dist == min_d, iota, N_EMB), axis=1)
    idx = idx.astype(jnp.int32)
    idx_ref[0, 0, :] = idx

    iota0 = jax.lax.broadcasted_iota(jnp.int32, (N_EMB, HW), 0)
    onehot_t = jnp.where(iota0 == idx[None, :], 1.0, 0.0).astype(jnp.float32)

    zq_t = jax.lax.dot_general(
        emb, onehot_t, (((0,), (0,)), ((), ())),
        preferred_element_type=jnp.float32,
        precision=jax.lax.Precision.HIGHEST)

    zq_ref[0] = x_slab + (zq_t - x_slab)

    diff = zq_t - x_slab
    part = jnp.sum(diff * diff, axis=0, keepdims=True)
    ones = jnp.full((HW, 1), 1.0, dtype=jnp.float32)
    cnt = jax.lax.dot_general(
        onehot_t, ones, (((1,), (0,)), ((), ())),
        preferred_element_type=jnp.float32)

    @pl.when(b == 0)
    def _init():
        loss_acc[...] = part
        counts_acc[...] = cnt

    @pl.when(b > 0)
    def _acc():
        loss_acc[...] += part
        counts_acc[...] += cnt

    @pl.when(b == B - 1)
    def _finalize():
        m = jnp.sum(loss_acc[...]) / jnp.float32(N_TOK * EMB_DIM)
        loss_ref[0, 0] = m + COMMITMENT_COST * m
        avg = counts_acc[...] / jnp.float32(N_TOK)
        ent = jnp.sum(avg * jnp.log(avg + 1e-10))
        perp_ref[0, 0] = jnp.exp(-ent)


def kernel(z_e, embedding):
    z3 = z_e.reshape(B, EMB_DIM, HW)

    dist, idx3, zq3, loss, perp = pl.pallas_call(
        _vq_block,
        grid=(B,),
        in_specs=[
            pl.BlockSpec((1, EMB_DIM, HW), lambda b: (b, 0, 0)),
            pl.BlockSpec((N_EMB, EMB_DIM), lambda b: (0, 0)),
        ],
        out_specs=[
            pl.BlockSpec((HW, N_EMB), lambda b: (b, 0)),
            pl.BlockSpec((1, 1, HW), lambda b: (b, 0, 0)),
            pl.BlockSpec((1, EMB_DIM, HW), lambda b: (b, 0, 0)),
            pl.BlockSpec(memory_space=pltpu.SMEM),
            pl.BlockSpec(memory_space=pltpu.SMEM),
        ],
        out_shape=[
            jax.ShapeDtypeStruct((N_TOK, N_EMB), jnp.float32),
            jax.ShapeDtypeStruct((B, 1, HW), jnp.int32),
            jax.ShapeDtypeStruct((B, EMB_DIM, HW), jnp.float32),
            jax.ShapeDtypeStruct((1, 1), jnp.float32),
            jax.ShapeDtypeStruct((1, 1), jnp.float32),
        ],
        scratch_shapes=[
            pltpu.VMEM((N_EMB, 1), jnp.float32),
            pltpu.VMEM((1, HW), jnp.float32),
        ],
        compiler_params=pltpu.CompilerParams(
            dimension_semantics=("arbitrary",)),
    )(z3, embedding)

    z_q_out = zq3.reshape(z_e.shape)
    encoding_indices = idx3.reshape(N_TOK)
    return (z_q_out, loss[0, 0], perp[0, 0], encoding_indices, dist)


# speedup vs baseline: 1.2918x; 1.2918x over previous
import jax
import jax.numpy as jnp
from jax.experimental import pallas as pl
from jax.experimental.pallas import tpu as pltpu

N_EMB = 1024
EMB_DIM = 64
B = 16
HW = 1024
N_TOK = B * HW
COMMITMENT_COST = 0.25


def _vq_block(z_ref, emb_ref, dist_ref, idx_ref, zq_ref, loss_ref, perp_ref,
              counts_acc, loss_acc):
    b = pl.program_id(0)

    x_slab = z_ref[0]
    emb = emb_ref[...]

    x2 = jnp.sum(x_slab * x_slab, axis=0, keepdims=True).T
    e2 = jnp.sum(emb * emb, axis=1, keepdims=True).T
    xe = jax.lax.dot_general(
        x_slab, emb, (((0,), (1,)), ((), ())),
        preferred_element_type=jnp.float32)
    dist = (x2 + e2) - 2.0 * xe
    dist_ref[...] = dist

    min_d = jnp.min(dist, axis=1, keepdims=True)
    iota = jax.lax.broadcasted_iota(jnp.int32, (HW, N_EMB), 1)
    idx = jnp.min(jnp.where(dist == min_d, iota, N_EMB), axis=1)
    idx = idx.astype(jnp.int32)
    idx_ref[0, 0, :] = idx

    iota0 = jax.lax.broadcasted_iota(jnp.int32, (N_EMB, HW), 0)
    onehot_t = jnp.where(iota0 == idx[None, :], 1.0, 0.0
                         ).astype(jnp.float32).astype(jnp.bfloat16)

    e_hi = emb.astype(jnp.bfloat16)
    r1 = emb - e_hi.astype(jnp.float32)
    e_mid = r1.astype(jnp.bfloat16)
    e_lo = (r1 - e_mid.astype(jnp.float32)).astype(jnp.bfloat16)

    def sel(e_part):
        return jax.lax.dot_general(
            e_part, onehot_t, (((0,), (0,)), ((), ())),
            preferred_element_type=jnp.float32)

    zq_t = (sel(e_hi) + sel(e_mid)) + sel(e_lo)

    zq_ref[0] = x_slab + (zq_t - x_slab)

    diff = zq_t - x_slab
    part = jnp.sum(diff * diff, axis=0, keepdims=True)
    ones = jnp.full((HW, 1), 1.0, dtype=jnp.bfloat16)
    cnt = jax.lax.dot_general(
        onehot_t, ones, (((1,), (0,)), ((), ())),
        preferred_element_type=jnp.float32)

    @pl.when(b == 0)
    def _init():
        loss_acc[...] = part
        counts_acc[...] = cnt

    @pl.when(b > 0)
    def _acc():
        loss_acc[...] += part
        counts_acc[...] += cnt

    @pl.when(b == B - 1)
    def _finalize():
        m = jnp.sum(loss_acc[...]) / jnp.float32(N_TOK * EMB_DIM)
        loss_ref[0, 0] = m + COMMITMENT_COST * m
        avg = counts_acc[...] / jnp.float32(N_TOK)
        ent = jnp.sum(avg * jnp.log(avg + 1e-10))
        perp_ref[0, 0] = jnp.exp(-ent)


def kernel(z_e, embedding):
    z3 = z_e.reshape(B, EMB_DIM, HW)

    dist, idx3, zq3, loss, perp = pl.pallas_call(
        _vq_block,
        grid=(B,),
        in_specs=[
            pl.BlockSpec((1, EMB_DIM, HW), lambda b: (b, 0, 0)),
            pl.BlockSpec((N_EMB, EMB_DIM), lambda b: (0, 0)),
        ],
        out_specs=[
            pl.BlockSpec((HW, N_EMB), lambda b: (b, 0)),
            pl.BlockSpec((1, 1, HW), lambda b: (b, 0, 0)),
            pl.BlockSpec((1, EMB_DIM, HW), lambda b: (b, 0, 0)),
            pl.BlockSpec(memory_space=pltpu.SMEM),
            pl.BlockSpec(memory_space=pltpu.SMEM),
        ],
        out_shape=[
            jax.ShapeDtypeStruct((N_TOK, N_EMB), jnp.float32),
            jax.ShapeDtypeStruct((B, 1, HW), jnp.int32),
            jax.ShapeDtypeStruct((B, EMB_DIM, HW), jnp.float32),
            jax.ShapeDtypeStruct((1, 1), jnp.float32),
            jax.ShapeDtypeStruct((1, 1), jnp.float32),
        ],
        scratch_shapes=[
            pltpu.VMEM((N_EMB, 1), jnp.float32),
            pltpu.VMEM((1, HW), jnp.float32),
        ],
        compiler_params=pltpu.CompilerParams(
            dimension_semantics=("arbitrary",)),
    )(z3, embedding)

    z_q_out = zq3.reshape(z_e.shape)
    encoding_indices = idx3.reshape(N_TOK)
    return (z_q_out, loss[0, 0], perp[0, 0], encoding_indices, dist)


# speedup vs baseline: 1.3855x; 1.0725x over previous
import jax
import jax.numpy as jnp
from jax.experimental import pallas as pl
from jax.experimental.pallas import tpu as pltpu

N_EMB = 1024
EMB_DIM = 64
B = 16
HW = 1024
N_TOK = B * HW
COMMITMENT_COST = 0.25


def _vq_block(z_ref, emb_ref, dist_ref, idx_ref, zq_ref, loss_ref, perp_ref,
              counts_acc, loss_acc):
    b = pl.program_id(0)

    x_slab = z_ref[0]
    emb = emb_ref[...]

    x2 = jnp.sum(x_slab * x_slab, axis=0, keepdims=True).T
    e2 = jnp.sum(emb * emb, axis=1, keepdims=True).T
    xe = jax.lax.dot_general(
        x_slab, emb, (((0,), (1,)), ((), ())),
        preferred_element_type=jnp.float32)
    dist = (x2 + e2) - 2.0 * xe
    dist_ref[...] = dist

    min_d = jnp.min(dist, axis=1, keepdims=True)
    iota = jax.lax.broadcasted_iota(jnp.int32, (HW, N_EMB), 1)
    idx = jnp.min(jnp.where(dist == min_d, iota, N_EMB), axis=1)
    idx = idx.astype(jnp.int32)
    idx_ref[0, 0, :] = idx

    iota0 = jax.lax.broadcasted_iota(jnp.int32, (N_EMB, HW), 0)
    onehot_t = jnp.where(iota0 == idx[None, :], 1.0, 0.0
                         ).astype(jnp.float32).astype(jnp.bfloat16)

    e_hi = emb.astype(jnp.bfloat16)
    r1 = emb - e_hi.astype(jnp.float32)
    e_mid = r1.astype(jnp.bfloat16)

    def sel(e_part):
        return jax.lax.dot_general(
            e_part, onehot_t, (((0,), (0,)), ((), ())),
            preferred_element_type=jnp.float32)

    zq_t = sel(e_hi) + sel(e_mid)

    zq_ref[0] = x_slab + (zq_t - x_slab)

    diff = zq_t - x_slab
    part = jnp.sum(diff * diff, axis=0, keepdims=True)
    ones = jnp.full((HW, 1), 1.0, dtype=jnp.bfloat16)
    cnt = jax.lax.dot_general(
        onehot_t, ones, (((1,), (0,)), ((), ())),
        preferred_element_type=jnp.float32)

    @pl.when(b == 0)
    def _init():
        loss_acc[...] = part
        counts_acc[...] = cnt

    @pl.when(b > 0)
    def _acc():
        loss_acc[...] += part
        counts_acc[...] += cnt

    @pl.when(b == B - 1)
    def _finalize():
        m = jnp.sum(loss_acc[...]) / jnp.float32(N_TOK * EMB_DIM)
        loss_ref[0, 0] = m + COMMITMENT_COST * m
        avg = counts_acc[...] / jnp.float32(N_TOK)
        ent = jnp.sum(avg * jnp.log(avg + 1e-10))
        perp_ref[0, 0] = jnp.exp(-ent)


def kernel(z_e, embedding):
    z3 = z_e.reshape(B, EMB_DIM, HW)

    dist, idx3, zq3, loss, perp = pl.pallas_call(
        _vq_block,
        grid=(B,),
        in_specs=[
            pl.BlockSpec((1, EMB_DIM, HW), lambda b: (b, 0, 0)),
            pl.BlockSpec((N_EMB, EMB_DIM), lambda b: (0, 0)),
        ],
        out_specs=[
            pl.BlockSpec((HW, N_EMB), lambda b: (b, 0)),
            pl.BlockSpec((1, 1, HW), lambda b: (b, 0, 0)),
            pl.BlockSpec((1, EMB_DIM, HW), lambda b: (b, 0, 0)),
            pl.BlockSpec(memory_space=pltpu.SMEM),
            pl.BlockSpec(memory_space=pltpu.SMEM),
        ],
        out_shape=[
            jax.ShapeDtypeStruct((N_TOK, N_EMB), jnp.float32),
            jax.ShapeDtypeStruct((B, 1, HW), jnp.int32),
            jax.ShapeDtypeStruct((B, EMB_DIM, HW), jnp.float32),
            jax.ShapeDtypeStruct((1, 1), jnp.float32),
            jax.ShapeDtypeStruct((1, 1), jnp.float32),
        ],
        scratch_shapes=[
            pltpu.VMEM((N_EMB, 1), jnp.float32),
            pltpu.VMEM((1, HW), jnp.float32),
        ],
        compiler_params=pltpu.CompilerParams(
            dimension_semantics=("arbitrary",)),
    )(z3, embedding)

    z_q_out = zq3.reshape(z_e.shape)
    encoding_indices = idx3.reshape(N_TOK)
    return (z_q_out, loss[0, 0], perp[0, 0], encoding_indices, dist)
